# baseline (speedup 1.0000x reference)
import functools

import jax
import jax.numpy as jnp
from jax import lax
from jax.experimental import pallas as pl
from jax.experimental.pallas import tpu as pltpu
from jax.experimental.pallas import tpu_sc as plsc

B = 16384
E = 36
N = 12
D = 128
LANES = 16
NC, NS = 2, 16
NW = NC * NS
PER_W = B // NW
CB = 8
G = PER_W // CB
XW = CB * E * D
OW = CB * N * D


def _sc_agg_body(x_hbm, w_hbm, o_hbm, wv, xv0, xv1, ov0, ov1, si0, si1, so0, so1):
    wid = lax.axis_index("s") * NC + lax.axis_index("c")
    base = wid * PER_W
    pltpu.sync_copy(w_hbm, wv)

    xvs, ovs, sis, sos = (xv0, xv1), (ov0, ov1), (si0, si1), (so0, so1)

    def in_copy(g, k):
        return pltpu.make_async_copy(
            x_hbm.at[pl.ds((base + g * CB) * (E * D), XW)], xvs[k], sis[k])

    def out_copy(g, k):
        return pltpu.make_async_copy(
            ovs[k], o_hbm.at[pl.ds((base + g * CB) * (N * D), OW)], sos[k])

    def compute(k):
        xv, ov = xvs[k], ovs[k]

        def body(i, _):
            xb = i * (E * D)
            ob = i * (N * D)
            for c in range(D // LANES):
                accs = [None] * N
                for e in range(E):
                    u, v = e // 6, 6 + e % 6
                    p = (xv[pl.ds(xb + e * D + c * LANES, LANES)]
                         * wv[pl.ds(e * LANES, LANES)])
                    accs[u] = p if accs[u] is None else accs[u] + p
                    accs[v] = p if accs[v] is None else accs[v] + p
                for n in range(N):
                    ov[pl.ds(ob + n * D + c * LANES, LANES)] = accs[n]
            return 0

        lax.fori_loop(0, CB, body, 0)

    in_copy(0, 0).start()
    in_copy(1, 1).start()

    def step(s, _):
        for k in range(2):
            g = s * 2 + k
            in_copy(g, k).wait()

            @pl.when(g >= 2)
            def _():
                out_copy(g - 2, k).wait()

            compute(k)
            out_copy(g, k).start()

            @pl.when(g + 2 < G)
            def _():
                in_copy(g + 2, k).start()
        return 0

    lax.fori_loop(0, G // 2, step, 0)
    out_copy(G - 2, 0).wait()
    out_copy(G - 1, 1).wait()


def _sc_aggregate(x_flat, w_bcast):
    mesh = plsc.VectorSubcoreMesh(
        core_axis_name="c", subcore_axis_name="s", num_cores=NC, num_subcores=NS)
    f = pl.kernel(
        _sc_agg_body,
        out_type=jax.ShapeDtypeStruct((B * N * D,), jnp.float32),
        mesh=mesh,
        scratch_types=[
            pltpu.VMEM((E * LANES,), jnp.float32),
            pltpu.VMEM((XW,), jnp.float32),
            pltpu.VMEM((XW,), jnp.float32),
            pltpu.VMEM((OW,), jnp.float32),
            pltpu.VMEM((OW,), jnp.float32),
            pltpu.SemaphoreType.DMA,
            pltpu.SemaphoreType.DMA,
            pltpu.SemaphoreType.DMA,
            pltpu.SemaphoreType.DMA,
        ],
    )
    return f(x_flat, w_bcast)


def _mm_body(x_ref, wt_ref, b_ref, o_ref):
    acc = jnp.dot(x_ref[:], wt_ref[:], preferred_element_type=jnp.float32)
    o_ref[:] = jnp.maximum(acc + b_ref[:], 0.0)


def _tc_linear(nf2d, wt, b2d):
    M = nf2d.shape[0]
    BM = 1024
    return pl.pallas_call(
        _mm_body,
        grid=(M // BM,),
        in_specs=[
            pl.BlockSpec((BM, D), lambda i: (i, 0)),
            pl.BlockSpec((D, D), lambda i: (0, 0)),
            pl.BlockSpec((1, D), lambda i: (0, 0)),
        ],
        out_specs=pl.BlockSpec((BM, D), lambda i: (i, 0)),
        out_shape=jax.ShapeDtypeStruct((M, D), jnp.float32),
        compiler_params=pltpu.CompilerParams(
            dimension_semantics=("arbitrary",)),
    )(nf2d, wt, b2d)


def kernel(edge_feats, edge_weights, W, b):
    w_sp = jax.nn.softplus(edge_weights.astype(jnp.float32))
    w_bcast = jnp.broadcast_to(w_sp[:, None], (E, LANES)).reshape(-1)
    x_flat = edge_feats.reshape(-1)
    nf_flat = _sc_aggregate(x_flat, w_bcast)
    nf2d = nf_flat.reshape(B * N, D)
    out = _tc_linear(nf2d, W.T, b.reshape(1, D))
    return out.reshape(B, N, D)


# speedup vs baseline: 1.8727x; 1.8727x over previous
import functools

import jax
import jax.numpy as jnp
from jax import lax
from jax.experimental import pallas as pl
from jax.experimental.pallas import tpu as pltpu
from jax.experimental.pallas import tpu_sc as plsc

B = 16384
E = 36
N = 12
D = 128
LANES = 16
NC, NS = 2, 16
NW = NC * NS
PER_W = B // NW
CB = 8
G = PER_W // CB


def _sc_agg_body(x_hbm, w_hbm, o_hbm, wv, xv0, xv1, ov0, ov1, si0, si1, so0, so1):
    wid = lax.axis_index("s") * NC + lax.axis_index("c")
    base = wid * PER_W
    pltpu.sync_copy(w_hbm, wv)

    xvs, ovs, sis, sos = (xv0, xv1), (ov0, ov1), (si0, si1), (so0, so1)

    def in_copy(g, k):
        return pltpu.make_async_copy(
            x_hbm.at[pl.ds(base + g * CB, CB)], xvs[k], sis[k])

    def out_copy(g, k):
        return pltpu.make_async_copy(
            ovs[k], o_hbm.at[pl.ds(base + g * CB, CB)], sos[k])

    def compute(k):
        xv, ov = xvs[k], ovs[k]

        def body(i, _):
            for c in range(D // LANES):
                sl = pl.ds(c * LANES, LANES)
                accs = [None] * N
                for e in range(E):
                    u, v = e // 6, 6 + e % 6
                    p = xv[i, e, sl] * wv[pl.ds(e * LANES, LANES)]
                    accs[u] = p if accs[u] is None else accs[u] + p
                    accs[v] = p if accs[v] is None else accs[v] + p
                for n in range(N):
                    ov[i, n, sl] = accs[n]
            return 0

        lax.fori_loop(0, CB, body, 0)

    in_copy(0, 0).start()
    in_copy(1, 1).start()

    def step(s, _):
        for k in range(2):
            g = s * 2 + k
            in_copy(g, k).wait()

            @pl.when(g >= 2)
            def _():
                out_copy(g - 2, k).wait()

            compute(k)
            out_copy(g, k).start()

            @pl.when(g + 2 < G)
            def _():
                in_copy(g + 2, k).start()
        return 0

    lax.fori_loop(0, G // 2, step, 0)
    out_copy(G - 2, 0).wait()
    out_copy(G - 1, 1).wait()


def _sc_aggregate(x, w_bcast):
    mesh = plsc.VectorSubcoreMesh(
        core_axis_name="c", subcore_axis_name="s", num_cores=NC, num_subcores=NS)
    f = pl.kernel(
        _sc_agg_body,
        out_type=jax.ShapeDtypeStruct((B, N, D), jnp.float32),
        mesh=mesh,
        scratch_types=[
            pltpu.VMEM((E * LANES,), jnp.float32),
            pltpu.VMEM((CB, E, D), jnp.float32),
            pltpu.VMEM((CB, E, D), jnp.float32),
            pltpu.VMEM((CB, N, D), jnp.float32),
            pltpu.VMEM((CB, N, D), jnp.float32),
            pltpu.SemaphoreType.DMA,
            pltpu.SemaphoreType.DMA,
            pltpu.SemaphoreType.DMA,
            pltpu.SemaphoreType.DMA,
        ],
        compiler_params=pltpu.CompilerParams(use_tc_tiling_on_sc=True),
    )
    return f(x, w_bcast)


def _mm_body(x_ref, wt_ref, b_ref, o_ref):
    acc = jax.lax.dot_general(
        x_ref[:], wt_ref[:], (((2,), (0,)), ((), ())),
        preferred_element_type=jnp.float32)
    o_ref[:] = jnp.maximum(acc + b_ref[:], 0.0)


def _tc_linear(nf, wt, b2d):
    BM = 1024
    return pl.pallas_call(
        _mm_body,
        grid=(B // BM,),
        in_specs=[
            pl.BlockSpec((BM, N, D), lambda i: (i, 0, 0)),
            pl.BlockSpec((D, D), lambda i: (0, 0)),
            pl.BlockSpec((1, D), lambda i: (0, 0)),
        ],
        out_specs=pl.BlockSpec((BM, N, D), lambda i: (i, 0, 0)),
        out_shape=jax.ShapeDtypeStruct((B, N, D), jnp.float32),
        compiler_params=pltpu.CompilerParams(
            dimension_semantics=("arbitrary",)),
    )(nf, wt, b2d)


def kernel(edge_feats, edge_weights, W, b):
    w_sp = jax.nn.softplus(edge_weights.astype(jnp.float32))
    w_bcast = jnp.broadcast_to(w_sp[:, None], (E, LANES)).reshape(-1)
    nf = _sc_aggregate(edge_feats, w_bcast)
    return _tc_linear(nf, W.T, b.reshape(1, D))


# speedup vs baseline: 3.0384x; 1.6225x over previous
import functools

import jax
import jax.numpy as jnp
from jax import lax
from jax.experimental import pallas as pl
from jax.experimental.pallas import tpu as pltpu
from jax.experimental.pallas import tpu_sc as plsc

B = 16384
E = 36
N = 12
D = 128
LANES = 16
NC, NS = 2, 16
NW = NC * NS
PER_W = B // NW
CB = 8
G = PER_W // CB


def _sc_agg_body(x_hbm, w_hbm, o_hbm, wv, xv0, xv1, ov0, ov1, si0, si1, so0, so1):
    wid = lax.axis_index("s") * NC + lax.axis_index("c")
    base = wid * PER_W
    pltpu.sync_copy(w_hbm, wv)

    xvs, ovs, sis, sos = (xv0, xv1), (ov0, ov1), (si0, si1), (so0, so1)

    def in_copy(g, k):
        return pltpu.make_async_copy(
            x_hbm.at[:, pl.ds(base + g * CB, CB), :], xvs[k], sis[k])

    def out_copy(g, k):
        return pltpu.make_async_copy(
            ovs[k], o_hbm.at[:, pl.ds(base + g * CB, CB), :], sos[k])

    def compute(k):
        xv, ov = xvs[k], ovs[k]

        def body(i, _):
            for c in range(D // LANES):
                sl = pl.ds(c * LANES, LANES)
                accs = [None] * N
                for e in range(E):
                    u, v = e // 6, 6 + e % 6
                    p = xv[e, i, sl] * wv[pl.ds(e * LANES, LANES)]
                    accs[u] = p if accs[u] is None else accs[u] + p
                    accs[v] = p if accs[v] is None else accs[v] + p
                for n in range(N):
                    ov[n, i, sl] = accs[n]
            return 0

        lax.fori_loop(0, CB, body, 0)

    in_copy(0, 0).start()
    in_copy(1, 1).start()

    def step(s, _):
        for k in range(2):
            g = s * 2 + k
            in_copy(g, k).wait()

            @pl.when(g >= 2)
            def _():
                out_copy(g - 2, k).wait()

            compute(k)
            out_copy(g, k).start()

            @pl.when(g + 2 < G)
            def _():
                in_copy(g + 2, k).start()
        return 0

    lax.fori_loop(0, G // 2, step, 0)
    out_copy(G - 2, 0).wait()
    out_copy(G - 1, 1).wait()


def _sc_aggregate(x_t, w_bcast):
    mesh = plsc.VectorSubcoreMesh(
        core_axis_name="c", subcore_axis_name="s", num_cores=NC, num_subcores=NS)
    f = pl.kernel(
        _sc_agg_body,
        out_type=jax.ShapeDtypeStruct((N, B, D), jnp.float32),
        mesh=mesh,
        scratch_types=[
            pltpu.VMEM((E * LANES,), jnp.float32),
            pltpu.VMEM((E, CB, D), jnp.float32),
            pltpu.VMEM((E, CB, D), jnp.float32),
            pltpu.VMEM((N, CB, D), jnp.float32),
            pltpu.VMEM((N, CB, D), jnp.float32),
            pltpu.SemaphoreType.DMA,
            pltpu.SemaphoreType.DMA,
            pltpu.SemaphoreType.DMA,
            pltpu.SemaphoreType.DMA,
        ],
        compiler_params=pltpu.CompilerParams(use_tc_tiling_on_sc=True),
    )
    return f(x_t, w_bcast)


def _mm_body(x_ref, wt_ref, b_ref, o_ref):
    acc = jnp.dot(x_ref[:], wt_ref[:], preferred_element_type=jnp.float32)
    o_ref[:] = jnp.maximum(acc + b_ref[:], 0.0)


def _tc_linear(nf2d, wt, b2d):
    M = nf2d.shape[0]
    BM = 1024
    return pl.pallas_call(
        _mm_body,
        grid=(M // BM,),
        in_specs=[
            pl.BlockSpec((BM, D), lambda i: (i, 0)),
            pl.BlockSpec((D, D), lambda i: (0, 0)),
            pl.BlockSpec((1, D), lambda i: (0, 0)),
        ],
        out_specs=pl.BlockSpec((BM, D), lambda i: (i, 0)),
        out_shape=jax.ShapeDtypeStruct((M, D), jnp.float32),
        compiler_params=pltpu.CompilerParams(
            dimension_semantics=("arbitrary",)),
    )(nf2d, wt, b2d)


def kernel(edge_feats, edge_weights, W, b):
    w_sp = jax.nn.softplus(edge_weights.astype(jnp.float32))
    w_bcast = jnp.broadcast_to(w_sp[:, None], (E, LANES)).reshape(-1)
    x_t = jnp.transpose(edge_feats, (1, 0, 2))
    nf_t = _sc_aggregate(x_t, w_bcast)
    o2d = _tc_linear(nf_t.reshape(N * B, D), W.T, b.reshape(1, D))
    return jnp.transpose(o2d.reshape(N, B, D), (1, 0, 2))


# speedup vs baseline: 4.0487x; 1.3325x over previous
import functools

import jax
import jax.numpy as jnp
from jax import lax
from jax.experimental import pallas as pl
from jax.experimental.pallas import tpu as pltpu
from jax.experimental.pallas import tpu_sc as plsc

B = 16384
E = 36
N = 12
D = 128
LANES = 16
NC, NS = 2, 16
NW = NC * NS
PER_W = B // NW
CB = 8
G = PER_W // CB


def _sc_agg_body(x_hbm, w_hbm, o_hbm, wv, xv0, xv1, ov0, ov1, si0, si1, so0, so1):
    wid = lax.axis_index("s") * NC + lax.axis_index("c")
    base = wid * PER_W
    pltpu.sync_copy(w_hbm, wv)

    xvs, ovs, sis, sos = (xv0, xv1), (ov0, ov1), (si0, si1), (so0, so1)

    def in_copy(g, k):
        return pltpu.make_async_copy(
            x_hbm.at[:, pl.ds(base + g * CB, CB), :], xvs[k], sis[k])

    def out_copy(g, k):
        return pltpu.make_async_copy(
            ovs[k], o_hbm.at[:, pl.ds(base + g * CB, CB), :], sos[k])

    def compute(k):
        xv, ov = xvs[k], ovs[k]

        def body(i, _):
            for c in range(D // LANES):
                sl = pl.ds(c * LANES, LANES)
                accs = [None] * N
                for e in range(E):
                    u, v = e // 6, 6 + e % 6
                    p = xv[e, i, sl] * wv[pl.ds(e * LANES, LANES)]
                    accs[u] = p if accs[u] is None else accs[u] + p
                    accs[v] = p if accs[v] is None else accs[v] + p
                for n in range(N):
                    ov[n, i, sl] = accs[n]
            return 0

        lax.fori_loop(0, CB, body, 0)

    in_copy(0, 0).start()
    in_copy(1, 1).start()

    def step(s, _):
        for k in range(2):
            g = s * 2 + k
            in_copy(g, k).wait()

            @pl.when(g >= 2)
            def _():
                out_copy(g - 2, k).wait()

            compute(k)
            out_copy(g, k).start()

            @pl.when(g + 2 < G)
            def _():
                in_copy(g + 2, k).start()
        return 0

    lax.fori_loop(0, G // 2, step, 0)
    out_copy(G - 2, 0).wait()
    out_copy(G - 1, 1).wait()


def _sc_aggregate(x_t, w_bcast):
    mesh = plsc.VectorSubcoreMesh(
        core_axis_name="c", subcore_axis_name="s", num_cores=NC, num_subcores=NS)
    f = pl.kernel(
        _sc_agg_body,
        out_type=jax.ShapeDtypeStruct((N, B, D), jnp.float32),
        mesh=mesh,
        scratch_types=[
            pltpu.VMEM((E * LANES,), jnp.float32),
            pltpu.VMEM((E, CB, D), jnp.float32),
            pltpu.VMEM((E, CB, D), jnp.float32),
            pltpu.VMEM((N, CB, D), jnp.float32),
            pltpu.VMEM((N, CB, D), jnp.float32),
            pltpu.SemaphoreType.DMA,
            pltpu.SemaphoreType.DMA,
            pltpu.SemaphoreType.DMA,
            pltpu.SemaphoreType.DMA,
        ],
        compiler_params=pltpu.CompilerParams(use_tc_tiling_on_sc=True),
    )
    return f(x_t, w_bcast)


def _mm_body(x_ref, wt_ref, b_ref, o_ref):
    acc = jnp.dot(x_ref[:], wt_ref[:], preferred_element_type=jnp.float32)
    o_ref[:] = jnp.maximum(acc + b_ref[:], 0.0)


def _tc_linear(nf2d, wt, b2d):
    M = nf2d.shape[0]
    BM = 8192
    return pl.pallas_call(
        _mm_body,
        grid=(M // BM,),
        in_specs=[
            pl.BlockSpec((BM, D), lambda i: (i, 0)),
            pl.BlockSpec((D, D), lambda i: (0, 0)),
            pl.BlockSpec((1, D), lambda i: (0, 0)),
        ],
        out_specs=pl.BlockSpec((BM, D), lambda i: (i, 0)),
        out_shape=jax.ShapeDtypeStruct((M, D), jnp.float32),
        compiler_params=pltpu.CompilerParams(
            dimension_semantics=("arbitrary",)),
    )(nf2d, wt, b2d)


def kernel(edge_feats, edge_weights, W, b):
    w_sp = jax.nn.softplus(edge_weights.astype(jnp.float32))
    w_bcast = jnp.broadcast_to(w_sp[:, None], (E, LANES)).reshape(-1)
    x_t = jnp.transpose(edge_feats, (1, 0, 2))
    nf_t = _sc_aggregate(x_t, w_bcast)
    o2d = _tc_linear(nf_t.reshape(N * B, D), W.T, b.reshape(1, D))
    return jnp.transpose(o2d.reshape(N, B, D), (1, 0, 2))
